# Initial kernel scaffold; baseline (speedup 1.0000x reference)
import jax
import jax.numpy as jnp
from jax.experimental import pallas as pl


def kernel(x, edge_index, W1, b1, Wg, att_src, att_dst, bg, g1, beta1, W2, b2, g2, beta2, W3, b3, W4, b4):
    raise NotImplementedError("write your pallas kernel here")


# speedup vs baseline: 11.9211x; 11.9211x over previous
import functools

import jax
import jax.numpy as jnp
from jax import lax
from jax.experimental import pallas as pl
from jax.experimental.pallas import tpu as pltpu
from jax.experimental.pallas import tpu_sc as plsc

N = 10000
NP = 10240
NPT = NP // 16
DIN = 128
HID = 64
HEADS = 4
GOUT = 128

ET = 330000
EP = 344064
EB = 128

_mesh = plsc.VectorSubcoreMesh(core_axis_name="c", subcore_axis_name="s")
_cp = pltpu.CompilerParams(use_tc_tiling_on_sc=False, needs_layout_passes=False)


def _add_offset(idx_ref, out_ref, off):
    for g in range(EB // 16):
        sl = pl.ds(g * 16, 16)
        out_ref[sl] = idx_ref[sl] + off


def _deg_call(dst_p, val8, z8):
    @functools.partial(
        pl.kernel,
        mesh=_mesh,
        compiler_params=_cp,
        out_type=jax.ShapeDtypeStruct((2, NP, 8), jnp.float32),
        scratch_types=[
            pltpu.VMEM((EB,), jnp.int32),
            pltpu.VMEM((EB, 8), jnp.float32),
            pltpu.VMEM_SHARED((NP, 8), jnp.float32),
        ],
    )
    def k(dst_hbm, val_hbm, z_hbm, out_hbm, dsti, val_v, acc):
        c = lax.axis_index("c")
        s = lax.axis_index("s")
        rows = pl.ds(s * NPT, NPT)
        pltpu.sync_copy(z_hbm.at[rows], acc.at[rows])
        pltpu.sync_copy(val_hbm, val_v)
        plsc.subcore_barrier()
        base = c * (EP // 2) + s * (EP // 32)

        def body(j, carry):
            pltpu.sync_copy(dst_hbm.at[pl.ds(base + j * EB, EB)], dsti)
            pltpu.sync_copy(val_v, acc.at[dsti], add=True)
            return carry

        lax.fori_loop(0, (EP // 32) // EB, body, 0)
        plsc.subcore_barrier()
        pltpu.sync_copy(acc.at[rows], out_hbm.at[c].at[rows])

    return k(dst_p, val8, z8)


def _gcn_pass(table_f, src_p, dst_p, zf, fh):
    @functools.partial(
        pl.kernel,
        mesh=_mesh,
        compiler_params=_cp,
        out_type=jax.ShapeDtypeStruct((2, NP, fh), jnp.float32),
        scratch_types=[
            pltpu.VMEM((EB,), jnp.int32),
            pltpu.VMEM((EB,), jnp.int32),
            pltpu.VMEM((EB,), jnp.int32),
            pltpu.VMEM((EB, fh), jnp.float32),
            pltpu.VMEM_SHARED((NP, fh), jnp.float32),
            pltpu.SemaphoreType.DMA,
        ],
    )
    def k(t_hbm, s_hbm, d_hbm, z_hbm, out_hbm, srci, srco, dsti, buf, acc, sem):
        c = lax.axis_index("c")
        s = lax.axis_index("s")
        rows = pl.ds(s * NPT, NPT)
        pltpu.sync_copy(z_hbm.at[rows], acc.at[rows])
        plsc.subcore_barrier()
        base = s * (EP // 16)
        off = c * NP

        def body(j, carry):
            e0 = base + j * EB
            pltpu.sync_copy(s_hbm.at[pl.ds(e0, EB)], srci)
            pltpu.sync_copy(d_hbm.at[pl.ds(e0, EB)], dsti)
            _add_offset(srci, srco, off)
            pltpu.async_copy(t_hbm.at[srco], buf, sem).wait()
            pltpu.sync_copy(buf, acc.at[dsti], add=True)
            return carry

        lax.fori_loop(0, (EP // 16) // EB, body, 0)
        plsc.subcore_barrier()
        pltpu.sync_copy(acc.at[rows], out_hbm.at[c].at[rows])

    return k(table_f, src_p, dst_p, zf)


GB = 64


def _gat_pass(g_f, asrc_t, adst_t, src_p, dst_p, z144):
    @functools.partial(
        pl.kernel,
        mesh=_mesh,
        compiler_params=_cp,
        out_type=jax.ShapeDtypeStruct((2, NP, 144), jnp.float32),
        scratch_types=[
            pltpu.VMEM((GB,), jnp.int32),
            pltpu.VMEM((GB,), jnp.int32),
            pltpu.VMEM((GB,), jnp.int32),
            pltpu.VMEM((GB,), jnp.int32),
            pltpu.VMEM((GB, 128), jnp.float32),
            pltpu.VMEM((GB, 16), jnp.float32),
            pltpu.VMEM((GB, 16), jnp.float32),
            pltpu.VMEM((GB, 144), jnp.float32),
            pltpu.VMEM((GB,), jnp.float32),
            pltpu.VMEM((GB,), jnp.float32),
            pltpu.VMEM_SHARED((NP, 144), jnp.float32),
            pltpu.SemaphoreType.DMA,
            pltpu.SemaphoreType.DMA,
            pltpu.SemaphoreType.DMA,
        ],
    )
    def k(g_hbm, as_hbm, ad_hbm, s_hbm, d_hbm, z_hbm, out_hbm,
          srci, srco, dsti, dsto, grows, asb, adb, orows, ex0, ex1,
          acc, sem0, sem1, sem2):
        c = lax.axis_index("c")
        s = lax.axis_index("s")
        rows = pl.ds(s * NPT, NPT)
        pltpu.sync_copy(z_hbm.at[rows], acc.at[rows])
        plsc.subcore_barrier()
        base = s * (EP // 16)
        off = c * NP
        lane = lax.iota(jnp.int32, 16)
        col0 = jnp.zeros((16,), jnp.int32)
        col1 = jnp.full((16,), 1, jnp.int32)
        col2 = jnp.full((16,), 2, jnp.int32)
        col3 = jnp.full((16,), 3, jnp.int32)

        def edge_body(i, carry):
            e0b = plsc.load_gather(ex0, [jnp.full((16,), i, jnp.int32)])
            e1b = plsc.load_gather(ex1, [jnp.full((16,), i, jnp.int32)])
            for f in range(4):
                sl = pl.ds(f * 16, 16)
                orows[i, sl] = grows[i, sl] * e0b
            for f in range(4, 8):
                sl = pl.ds(f * 16, 16)
                orows[i, sl] = grows[i, sl] * e1b
            tail = jnp.where(lane == 0, e0b,
                             jnp.where(lane == 1, e1b, jnp.float32(0)))
            orows[i, pl.ds(128, 16)] = tail
            return carry

        def body(j, carry):
            e0 = base + j * GB
            pltpu.sync_copy(s_hbm.at[pl.ds(e0, GB)], srci)
            pltpu.sync_copy(d_hbm.at[pl.ds(e0, GB)], dsti)
            for g in range(GB // 16):
                sl = pl.ds(g * 16, 16)
                srco[sl] = srci[sl] + off
                dsto[sl] = dsti[sl] + off
            cp0 = pltpu.async_copy(g_hbm.at[srco], grows, sem0)
            cp1 = pltpu.async_copy(as_hbm.at[srco], asb, sem1)
            cp2 = pltpu.async_copy(ad_hbm.at[dsto], adb, sem2)
            cp0.wait()
            cp1.wait()
            cp2.wait()
            for g in range(GB // 16):
                sl = pl.ds(g * 16, 16)
                r16 = jnp.full((16,), g * 16, jnp.int32) + lane
                a0 = plsc.load_gather(asb, [r16, col0])
                a1 = plsc.load_gather(asb, [r16, col1])
                b0 = plsc.load_gather(adb, [r16, col0])
                b1 = plsc.load_gather(adb, [r16, col1])
                m0 = plsc.load_gather(adb, [r16, col2])
                m1 = plsc.load_gather(adb, [r16, col3])
                e0v = a0 + b0
                e0v = jnp.where(e0v > 0, e0v, 0.2 * e0v)
                e1v = a1 + b1
                e1v = jnp.where(e1v > 0, e1v, 0.2 * e1v)
                ex0[sl] = jnp.exp(e0v - m0)
                ex1[sl] = jnp.exp(e1v - m1)
            lax.fori_loop(0, GB, edge_body, 0)
            pltpu.sync_copy(orows, acc.at[dsti], add=True)
            return carry

        lax.fori_loop(0, (EP // 16) // GB, body, 0)
        plsc.subcore_barrier()
        pltpu.sync_copy(acc.at[rows], out_hbm.at[c].at[rows])

    return k(g_f, asrc_t, adst_t, src_p, dst_p, z144)


def _tc_a(degp, x_p, W1):
    def body(dp, xr, w1, hs_o, dinv_o):
        deg = dp[0, :, 0:1] + dp[1, :, 0:1]
        dinv = jnp.where(deg > 0, lax.rsqrt(jnp.maximum(deg, 1.0)), 0.0)
        h = jnp.dot(xr[...], w1[...], preferred_element_type=jnp.float32)
        hs_o[...] = h * dinv
        dinv_o[...] = jnp.broadcast_to(dinv, (NP, 8))

    return pl.pallas_call(
        body,
        out_shape=[
            jax.ShapeDtypeStruct((NP, HID), jnp.float32),
            jax.ShapeDtypeStruct((NP, 8), jnp.float32),
        ],
    )(degp, x_p, W1)


def _tc_b(S1, dinv8, b1, Wg, As, Ad):
    def body(s1, dv, b1r, wg, asr, adr, g_o, as_o, ad_o, mp_o):
        S = jnp.concatenate([s1[0], s1[1]], axis=1)
        dinv = dv[:, 0:1]
        h1 = jnp.maximum(S * dinv + b1r[...], 0.0)
        g = jnp.dot(h1, wg[...], preferred_element_type=jnp.float32)
        g_o[...] = g
        a_s = jnp.dot(g, asr[...], preferred_element_type=jnp.float32)
        a_d = jnp.dot(g, adr[...], preferred_element_type=jnp.float32)
        as_o[...] = a_s
        ad_o[...] = a_d
        mg = jnp.max(a_s, axis=0, keepdims=True)
        t = mg + a_d
        mp_o[...] = jnp.where(t > 0, t, 0.2 * t)

    return pl.pallas_call(
        body,
        out_shape=[
            jax.ShapeDtypeStruct((NP, HEADS * HID), jnp.float32),
            jax.ShapeDtypeStruct((NP, HEADS), jnp.float32),
            jax.ShapeDtypeStruct((NP, HEADS), jnp.float32),
            jax.ShapeDtypeStruct((NP, HEADS), jnp.float32),
        ],
    )(S1, dinv8, b1, Wg, As, Ad)


def _ln(h, g, b):
    mu = jnp.mean(h, axis=-1, keepdims=True)
    var = jnp.mean((h - mu) ** 2, axis=-1, keepdims=True)
    return (h - mu) * lax.rsqrt(var + 1e-5) * g + b


def _tc_c(accg, bg, g1, beta1, W2, dinv8):
    def body(ac, bgr, g1r, be1, w2, dv, hs_o):
        pieces = []
        for c in range(2):
            num = ac[c, :, 0:128]
            den0 = ac[c, :, 128:129]
            den1 = ac[c, :, 129:130]
            pieces.append(num[:, 0:64] / (den0 + 1e-16))
            pieces.append(num[:, 64:128] / (den1 + 1e-16))
        gat = jnp.concatenate(pieces, axis=1) + bgr[...]
        h2 = _ln(gat, g1r[...], be1[...])
        hh = jnp.dot(h2, w2[...], preferred_element_type=jnp.float32)
        hs_o[...] = hh * dv[:, 0:1]

    return pl.pallas_call(
        body,
        out_shape=jax.ShapeDtypeStruct((NP, GOUT), jnp.float32),
    )(accg, bg, g1, beta1, W2, dinv8)


def _tc_d(S2, dinv8, b2, g2, beta2, x_p, W3, b3, W4, b4):
    def body(s2, dv, b2r, g2r, be2, xr, w3, b3r, w4, b4r, o):
        S = jnp.concatenate([s2[0], s2[1]], axis=1)
        h3 = S * dv[:, 0:1] + b2r[...]
        h3 = jnp.maximum(_ln(h3, g2r[...], be2[...]), 0.0)
        hc = jnp.concatenate([h3, xr[...]], axis=1)
        h4 = jnp.maximum(
            jnp.dot(hc, w3[...], preferred_element_type=jnp.float32) + b3r[...],
            0.0)
        o[...] = jnp.dot(h4, w4[...], preferred_element_type=jnp.float32) + b4r[...]

    return pl.pallas_call(
        body,
        out_shape=jax.ShapeDtypeStruct((NP, 2), jnp.float32),
    )(S2, dinv8, b2, g2, beta2, x_p, W3, b3, W4, b4)


def kernel(x, edge_index, W1, b1, Wg, att_src, att_dst, bg, g1, beta1,
           W2, b2, g2, beta2, W3, b3, W4, b4):
    loop = jnp.arange(N, dtype=jnp.int32)
    src = jnp.concatenate([edge_index[0], loop])
    dst = jnp.concatenate([edge_index[1], loop])
    pad = jnp.full((EP - ET,), NP - 1, jnp.int32)
    src_p = jnp.concatenate([src, pad])
    dst_p = jnp.concatenate([dst, pad])
    x_p = jnp.pad(x, ((0, NP - N), (0, 0)))

    val8 = jnp.concatenate(
        [jnp.ones((EB, 1), jnp.float32), jnp.zeros((EB, 7), jnp.float32)], axis=1)
    z8 = jnp.zeros((NP, 8), jnp.float32)
    z32 = jnp.zeros((NP, 32), jnp.float32)
    z64 = jnp.zeros((NP, 64), jnp.float32)
    z144 = jnp.zeros((NP, 144), jnp.float32)

    degp = _deg_call(dst_p, val8, z8)
    hs1, dinv8 = _tc_a(degp, x_p, W1)
    hs1f = jnp.concatenate([hs1[:, :32], hs1[:, 32:]], axis=0)
    S1 = _gcn_pass(hs1f, src_p, dst_p, z32, 32)

    head = jnp.arange(HEADS * HID, dtype=jnp.int32) // HID
    sel = (head[:, None] == jnp.arange(HEADS, dtype=jnp.int32)[None, :])
    As = jnp.where(sel, att_src.reshape(-1)[:, None], 0.0)
    Ad = jnp.where(sel, att_dst.reshape(-1)[:, None], 0.0)

    g, a_s, a_d, mp = _tc_b(S1, dinv8, b1, Wg, As, Ad)
    gf = jnp.concatenate([g[:, :128], g[:, 128:]], axis=0)
    asrc_t = jnp.concatenate([
        jnp.pad(a_s[:, 0:2], ((0, 0), (0, 14))),
        jnp.pad(a_s[:, 2:4], ((0, 0), (0, 14))),
    ], axis=0)
    adst_t = jnp.concatenate([
        jnp.pad(jnp.concatenate([a_d[:, 0:2], mp[:, 0:2]], axis=1),
                ((0, 0), (0, 12))),
        jnp.pad(jnp.concatenate([a_d[:, 2:4], mp[:, 2:4]], axis=1),
                ((0, 0), (0, 12))),
    ], axis=0)
    accg = _gat_pass(gf, asrc_t, adst_t, src_p, dst_p, z144)

    hs2 = _tc_c(accg, bg, g1, beta1, W2, dinv8)
    hs2f = jnp.concatenate([hs2[:, :64], hs2[:, 64:]], axis=0)
    S2 = _gcn_pass(hs2f, src_p, dst_p, z64, 64)

    out = _tc_d(S2, dinv8, b2, g2, beta2, x_p, W3, b3, W4, b4)
    return out[:N]


# speedup vs baseline: 17.1256x; 1.4366x over previous
import functools

import jax
import jax.numpy as jnp
from jax import lax
from jax.experimental import pallas as pl
from jax.experimental.pallas import tpu as pltpu
from jax.experimental.pallas import tpu_sc as plsc

N = 10000
NP = 10240
NPT = NP // 16
DIN = 128
HID = 64
HEADS = 4
GOUT = 128

ET = 330000
EP = 344064
EB = 128

_mesh = plsc.VectorSubcoreMesh(core_axis_name="c", subcore_axis_name="s")
_cp = pltpu.CompilerParams(use_tc_tiling_on_sc=False, needs_layout_passes=False)


def _add_offset(idx_ref, out_ref, off):
    for g in range(EB // 16):
        sl = pl.ds(g * 16, 16)
        out_ref[sl] = idx_ref[sl] + off


def _deg_call(dst_p, val8, z8):
    @functools.partial(
        pl.kernel,
        mesh=_mesh,
        compiler_params=_cp,
        out_type=jax.ShapeDtypeStruct((2, NP, 8), jnp.float32),
        scratch_types=[
            pltpu.VMEM((EB,), jnp.int32),
            pltpu.VMEM((EB, 8), jnp.float32),
            pltpu.VMEM_SHARED((NP, 8), jnp.float32),
        ],
    )
    def k(dst_hbm, val_hbm, z_hbm, out_hbm, dsti, val_v, acc):
        c = lax.axis_index("c")
        s = lax.axis_index("s")
        rows = pl.ds(s * NPT, NPT)
        pltpu.sync_copy(z_hbm.at[rows], acc.at[rows])
        pltpu.sync_copy(val_hbm, val_v)
        plsc.subcore_barrier()
        base = c * (EP // 2) + s * (EP // 32)

        def body(j, carry):
            pltpu.sync_copy(dst_hbm.at[pl.ds(base + j * EB, EB)], dsti)
            pltpu.sync_copy(val_v, acc.at[dsti], add=True)
            return carry

        lax.fori_loop(0, (EP // 32) // EB, body, 0)
        plsc.subcore_barrier()
        pltpu.sync_copy(acc.at[rows], out_hbm.at[c].at[rows])

    return k(dst_p, val8, z8)


def _gcn_pass(table_f, src_p, dst_p, zf, fh):
    @functools.partial(
        pl.kernel,
        mesh=_mesh,
        compiler_params=_cp,
        out_type=jax.ShapeDtypeStruct((2, NP, fh), jnp.float32),
        scratch_types=[
            pltpu.VMEM((EB,), jnp.int32),
            pltpu.VMEM((EB,), jnp.int32),
            pltpu.VMEM((EB,), jnp.int32),
            pltpu.VMEM((EB, fh), jnp.float32),
            pltpu.VMEM_SHARED((NP, fh), jnp.float32),
            pltpu.SemaphoreType.DMA,
        ],
    )
    def k(t_hbm, s_hbm, d_hbm, z_hbm, out_hbm, srci, srco, dsti, buf, acc, sem):
        c = lax.axis_index("c")
        s = lax.axis_index("s")
        rows = pl.ds(s * NPT, NPT)
        pltpu.sync_copy(z_hbm.at[rows], acc.at[rows])
        plsc.subcore_barrier()
        base = s * (EP // 16)
        off = c * NP

        def body(j, carry):
            e0 = base + j * EB
            pltpu.sync_copy(s_hbm.at[pl.ds(e0, EB)], srci)
            pltpu.sync_copy(d_hbm.at[pl.ds(e0, EB)], dsti)
            _add_offset(srci, srco, off)
            pltpu.async_copy(t_hbm.at[srco], buf, sem).wait()
            pltpu.sync_copy(buf, acc.at[dsti], add=True)
            return carry

        lax.fori_loop(0, (EP // 16) // EB, body, 0)
        plsc.subcore_barrier()
        pltpu.sync_copy(acc.at[rows], out_hbm.at[c].at[rows])

    return k(table_f, src_p, dst_p, zf)


GB = 64


def _gat_pass(g_f, adst_t, src_p, dst_p, z144):
    NBLK = (EP // 16) // GB

    @functools.partial(
        pl.kernel,
        mesh=_mesh,
        compiler_params=_cp,
        out_type=jax.ShapeDtypeStruct((2, NP, 144), jnp.float32),
        scratch_types=[
            pltpu.VMEM((GB,), jnp.int32),
            pltpu.VMEM((GB,), jnp.int32),
            pltpu.VMEM((GB,), jnp.int32),
            pltpu.VMEM((GB,), jnp.int32),
            pltpu.VMEM((GB,), jnp.int32),
            pltpu.VMEM((GB, 144), jnp.float32),
            pltpu.VMEM((GB, 144), jnp.float32),
            pltpu.VMEM((GB, 16), jnp.float32),
            pltpu.VMEM((GB,), jnp.float32),
            pltpu.VMEM((GB,), jnp.float32),
            pltpu.VMEM_SHARED((NP, 144), jnp.float32),
            pltpu.SemaphoreType.DMA,
            pltpu.SemaphoreType.DMA,
            pltpu.SemaphoreType.DMA,
            pltpu.SemaphoreType.DMA,
        ],
    )
    def k(g_hbm, ad_hbm, s_hbm, d_hbm, z_hbm, out_hbm,
          srci, srco, dsto, dsti0, dsti1, orows0, orows1, adb, ex0, ex1,
          acc, semg, sema, sems0, sems1):
        c = lax.axis_index("c")
        s = lax.axis_index("s")
        rows = pl.ds(s * NPT, NPT)
        pltpu.sync_copy(z_hbm.at[rows], acc.at[rows])
        plsc.subcore_barrier()
        base = s * (EP // 16)
        off = c * NP
        lane = lax.iota(jnp.int32, 16)
        col0 = jnp.zeros((16,), jnp.int32)
        col1 = jnp.full((16,), 1, jnp.int32)
        col2 = jnp.full((16,), 2, jnp.int32)
        col3 = jnp.full((16,), 3, jnp.int32)
        col128 = jnp.full((16,), 128, jnp.int32)
        col129 = jnp.full((16,), 129, jnp.int32)

        def load_compute(j, dsti, orows):
            e0 = base + j * GB
            pltpu.sync_copy(s_hbm.at[pl.ds(e0, GB)], srci)
            pltpu.sync_copy(d_hbm.at[pl.ds(e0, GB)], dsti)
            for g in range(GB // 16):
                sl = pl.ds(g * 16, 16)
                srco[sl] = srci[sl] + off
                dsto[sl] = dsti[sl] + off
            cp0 = pltpu.async_copy(g_hbm.at[srco], orows, semg)
            cp1 = pltpu.async_copy(ad_hbm.at[dsto], adb, sema)
            cp0.wait()
            cp1.wait()
            for g in range(GB // 16):
                sl = pl.ds(g * 16, 16)
                r16 = jnp.full((16,), g * 16, jnp.int32) + lane
                a0 = plsc.load_gather(orows, [r16, col128])
                a1 = plsc.load_gather(orows, [r16, col129])
                b0 = plsc.load_gather(adb, [r16, col0])
                b1 = plsc.load_gather(adb, [r16, col1])
                m0 = plsc.load_gather(adb, [r16, col2])
                m1 = plsc.load_gather(adb, [r16, col3])
                e0v = a0 + b0
                e0v = jnp.where(e0v > 0, e0v, 0.2 * e0v)
                e1v = a1 + b1
                e1v = jnp.where(e1v > 0, e1v, 0.2 * e1v)
                ex0[sl] = jnp.exp(e0v - m0)
                ex1[sl] = jnp.exp(e1v - m1)

            def edge_body(i, carry):
                e0b = plsc.load_gather(ex0, [jnp.full((16,), i, jnp.int32)])
                e1b = plsc.load_gather(ex1, [jnp.full((16,), i, jnp.int32)])
                for f in range(4):
                    sl = pl.ds(f * 16, 16)
                    orows[i, sl] = orows[i, sl] * e0b
                for f in range(4, 8):
                    sl = pl.ds(f * 16, 16)
                    orows[i, sl] = orows[i, sl] * e1b
                tail = jnp.where(lane == 0, e0b,
                                 jnp.where(lane == 1, e1b, jnp.float32(0)))
                orows[i, pl.ds(128, 16)] = tail
                return carry

            lax.fori_loop(0, GB, edge_body, 0)

        load_compute(0, dsti0, orows0)
        pltpu.async_copy(orows0, acc.at[dsti0], sems0, add=True)
        load_compute(1, dsti1, orows1)
        pltpu.async_copy(orows1, acc.at[dsti1], sems1, add=True)

        def body(k2, carry):
            j = 2 + 2 * k2
            pltpu.make_async_copy(orows0, acc.at[dsti0], sems0).wait()
            load_compute(j, dsti0, orows0)
            pltpu.async_copy(orows0, acc.at[dsti0], sems0, add=True)
            pltpu.make_async_copy(orows1, acc.at[dsti1], sems1).wait()
            load_compute(j + 1, dsti1, orows1)
            pltpu.async_copy(orows1, acc.at[dsti1], sems1, add=True)
            return carry

        lax.fori_loop(0, (NBLK - 2) // 2, body, 0)
        pltpu.make_async_copy(orows0, acc.at[dsti0], sems0).wait()
        pltpu.make_async_copy(orows1, acc.at[dsti1], sems1).wait()
        plsc.subcore_barrier()
        pltpu.sync_copy(acc.at[rows], out_hbm.at[c].at[rows])

    return k(g_f, adst_t, src_p, dst_p, z144)


def _tc_a(degp, x_p, W1):
    def body(dp, xr, w1, hs_o, dinv_o):
        deg = dp[0, :, 0:1] + dp[1, :, 0:1]
        dinv = jnp.where(deg > 0, lax.rsqrt(jnp.maximum(deg, 1.0)), 0.0)
        h = jnp.dot(xr[...], w1[...], preferred_element_type=jnp.float32)
        hs_o[...] = h * dinv
        dinv_o[...] = jnp.broadcast_to(dinv, (NP, 8))

    return pl.pallas_call(
        body,
        out_shape=[
            jax.ShapeDtypeStruct((NP, HID), jnp.float32),
            jax.ShapeDtypeStruct((NP, 8), jnp.float32),
        ],
    )(degp, x_p, W1)


def _tc_b(S1, dinv8, b1, Wg, As, Ad):
    def body(s1, dv, b1r, wg, asr, adr, g_o, as_o, ad_o, mp_o):
        S = jnp.concatenate([s1[0], s1[1]], axis=1)
        dinv = dv[:, 0:1]
        h1 = jnp.maximum(S * dinv + b1r[...], 0.0)
        g = jnp.dot(h1, wg[...], preferred_element_type=jnp.float32)
        g_o[...] = g
        a_s = jnp.dot(g, asr[...], preferred_element_type=jnp.float32)
        a_d = jnp.dot(g, adr[...], preferred_element_type=jnp.float32)
        as_o[...] = a_s
        ad_o[...] = a_d
        mg = jnp.max(a_s, axis=0, keepdims=True)
        t = mg + a_d
        mp_o[...] = jnp.where(t > 0, t, 0.2 * t)

    return pl.pallas_call(
        body,
        out_shape=[
            jax.ShapeDtypeStruct((NP, HEADS * HID), jnp.float32),
            jax.ShapeDtypeStruct((NP, HEADS), jnp.float32),
            jax.ShapeDtypeStruct((NP, HEADS), jnp.float32),
            jax.ShapeDtypeStruct((NP, HEADS), jnp.float32),
        ],
    )(S1, dinv8, b1, Wg, As, Ad)


def _ln(h, g, b):
    mu = jnp.mean(h, axis=-1, keepdims=True)
    var = jnp.mean((h - mu) ** 2, axis=-1, keepdims=True)
    return (h - mu) * lax.rsqrt(var + 1e-5) * g + b


def _tc_c(accg, bg, g1, beta1, W2, dinv8):
    def body(ac, bgr, g1r, be1, w2, dv, hs_o):
        pieces = []
        for c in range(2):
            num = ac[c, :, 0:128]
            den0 = ac[c, :, 128:129]
            den1 = ac[c, :, 129:130]
            pieces.append(num[:, 0:64] / (den0 + 1e-16))
            pieces.append(num[:, 64:128] / (den1 + 1e-16))
        gat = jnp.concatenate(pieces, axis=1) + bgr[...]
        h2 = _ln(gat, g1r[...], be1[...])
        hh = jnp.dot(h2, w2[...], preferred_element_type=jnp.float32)
        hs_o[...] = hh * dv[:, 0:1]

    return pl.pallas_call(
        body,
        out_shape=jax.ShapeDtypeStruct((NP, GOUT), jnp.float32),
    )(accg, bg, g1, beta1, W2, dinv8)


def _tc_d(S2, dinv8, b2, g2, beta2, x_p, W3, b3, W4, b4):
    def body(s2, dv, b2r, g2r, be2, xr, w3, b3r, w4, b4r, o):
        S = jnp.concatenate([s2[0], s2[1]], axis=1)
        h3 = S * dv[:, 0:1] + b2r[...]
        h3 = jnp.maximum(_ln(h3, g2r[...], be2[...]), 0.0)
        hc = jnp.concatenate([h3, xr[...]], axis=1)
        h4 = jnp.maximum(
            jnp.dot(hc, w3[...], preferred_element_type=jnp.float32) + b3r[...],
            0.0)
        o[...] = jnp.dot(h4, w4[...], preferred_element_type=jnp.float32) + b4r[...]

    return pl.pallas_call(
        body,
        out_shape=jax.ShapeDtypeStruct((NP, 2), jnp.float32),
    )(S2, dinv8, b2, g2, beta2, x_p, W3, b3, W4, b4)


def kernel(x, edge_index, W1, b1, Wg, att_src, att_dst, bg, g1, beta1,
           W2, b2, g2, beta2, W3, b3, W4, b4):
    loop = jnp.arange(N, dtype=jnp.int32)
    src = jnp.concatenate([edge_index[0], loop])
    dst = jnp.concatenate([edge_index[1], loop])
    pad = jnp.full((EP - ET,), NP - 1, jnp.int32)
    src_p = jnp.concatenate([src, pad])
    dst_p = jnp.concatenate([dst, pad])
    x_p = jnp.pad(x, ((0, NP - N), (0, 0)))

    val8 = jnp.concatenate(
        [jnp.ones((EB, 1), jnp.float32), jnp.zeros((EB, 7), jnp.float32)], axis=1)
    z8 = jnp.zeros((NP, 8), jnp.float32)
    z32 = jnp.zeros((NP, 32), jnp.float32)
    z64 = jnp.zeros((NP, 64), jnp.float32)
    z144 = jnp.zeros((NP, 144), jnp.float32)

    degp = _deg_call(dst_p, val8, z8)
    hs1, dinv8 = _tc_a(degp, x_p, W1)
    hs1f = jnp.concatenate([hs1[:, :32], hs1[:, 32:]], axis=0)
    S1 = _gcn_pass(hs1f, src_p, dst_p, z32, 32)

    head = jnp.arange(HEADS * HID, dtype=jnp.int32) // HID
    sel = (head[:, None] == jnp.arange(HEADS, dtype=jnp.int32)[None, :])
    As = jnp.where(sel, att_src.reshape(-1)[:, None], 0.0)
    Ad = jnp.where(sel, att_dst.reshape(-1)[:, None], 0.0)

    g, a_s, a_d, mp = _tc_b(S1, dinv8, b1, Wg, As, Ad)
    gf = jnp.concatenate([
        jnp.pad(jnp.concatenate([g[:, :128], a_s[:, 0:2]], axis=1),
                ((0, 0), (0, 14))),
        jnp.pad(jnp.concatenate([g[:, 128:], a_s[:, 2:4]], axis=1),
                ((0, 0), (0, 14))),
    ], axis=0)
    adst_t = jnp.concatenate([
        jnp.pad(jnp.concatenate([a_d[:, 0:2], mp[:, 0:2]], axis=1),
                ((0, 0), (0, 12))),
        jnp.pad(jnp.concatenate([a_d[:, 2:4], mp[:, 2:4]], axis=1),
                ((0, 0), (0, 12))),
    ], axis=0)
    accg = _gat_pass(gf, adst_t, src_p, dst_p, z144)

    hs2 = _tc_c(accg, bg, g1, beta1, W2, dinv8)
    hs2f = jnp.concatenate([hs2[:, :64], hs2[:, 64:]], axis=0)
    S2 = _gcn_pass(hs2f, src_p, dst_p, z64, 64)

    out = _tc_d(S2, dinv8, b2, g2, beta2, x_p, W3, b3, W4, b4)
    return out[:N]


# speedup vs baseline: 19.8863x; 1.1612x over previous
import functools

import jax
import jax.numpy as jnp
from jax import lax
from jax.experimental import pallas as pl
from jax.experimental.pallas import tpu as pltpu
from jax.experimental.pallas import tpu_sc as plsc

N = 10000
NP = 10240
NPT = NP // 16
DIN = 128
HID = 64
HEADS = 4
GOUT = 128

ET = 330000
EP = 344064
EB = 128

_mesh = plsc.VectorSubcoreMesh(core_axis_name="c", subcore_axis_name="s")
_cp = pltpu.CompilerParams(use_tc_tiling_on_sc=False, needs_layout_passes=False)


def _add_offset(idx_ref, out_ref, off):
    for g in range(EB // 16):
        sl = pl.ds(g * 16, 16)
        out_ref[sl] = idx_ref[sl] + off


def _deg_call(dst_p, val8, z8):
    @functools.partial(
        pl.kernel,
        mesh=_mesh,
        compiler_params=_cp,
        out_type=jax.ShapeDtypeStruct((2, NP, 8), jnp.float32),
        scratch_types=[
            pltpu.VMEM((EB,), jnp.int32),
            pltpu.VMEM((EB, 8), jnp.float32),
            pltpu.VMEM_SHARED((NP, 8), jnp.float32),
        ],
    )
    def k(dst_hbm, val_hbm, z_hbm, out_hbm, dsti, val_v, acc):
        c = lax.axis_index("c")
        s = lax.axis_index("s")
        rows = pl.ds(s * NPT, NPT)
        pltpu.sync_copy(z_hbm.at[rows], acc.at[rows])
        pltpu.sync_copy(val_hbm, val_v)
        plsc.subcore_barrier()
        base = c * (EP // 2) + s * (EP // 32)

        def body(j, carry):
            pltpu.sync_copy(dst_hbm.at[pl.ds(base + j * EB, EB)], dsti)
            pltpu.sync_copy(val_v, acc.at[dsti], add=True)
            return carry

        lax.fori_loop(0, (EP // 32) // EB, body, 0)
        plsc.subcore_barrier()
        pltpu.sync_copy(acc.at[rows], out_hbm.at[c].at[rows])

    return k(dst_p, val8, z8)


def _gcn_pass(table_f, src_p, dst_p, zf, fh):
    NB = (EP // 16) // EB

    @functools.partial(
        pl.kernel,
        mesh=_mesh,
        compiler_params=_cp,
        out_type=jax.ShapeDtypeStruct((2, NP, fh), jnp.float32),
        scratch_types=[
            pltpu.VMEM((EB,), jnp.int32),
            pltpu.VMEM((EB,), jnp.int32),
            pltpu.VMEM((EB,), jnp.int32),
            pltpu.VMEM((EB, fh), jnp.float32),
            pltpu.VMEM((EB,), jnp.int32),
            pltpu.VMEM((EB,), jnp.int32),
            pltpu.VMEM((EB,), jnp.int32),
            pltpu.VMEM((EB, fh), jnp.float32),
            pltpu.VMEM_SHARED((NP, fh), jnp.float32),
            pltpu.SemaphoreType.DMA,
            pltpu.SemaphoreType.DMA,
            pltpu.SemaphoreType.DMA,
            pltpu.SemaphoreType.DMA,
        ],
    )
    def k(t_hbm, s_hbm, d_hbm, z_hbm, out_hbm,
          srci0, srco0, dsti0, buf0, srci1, srco1, dsti1, buf1,
          acc, semg0, semg1, sems0, sems1):
        c = lax.axis_index("c")
        s = lax.axis_index("s")
        rows = pl.ds(s * NPT, NPT)
        pltpu.sync_copy(z_hbm.at[rows], acc.at[rows])
        plsc.subcore_barrier()
        base = s * (EP // 16)
        off = c * NP

        def load(j, srci, srco, dsti, buf, semg):
            e0 = base + j * EB
            pltpu.sync_copy(s_hbm.at[pl.ds(e0, EB)], srci)
            pltpu.sync_copy(d_hbm.at[pl.ds(e0, EB)], dsti)
            _add_offset(srci, srco, off)
            pltpu.async_copy(t_hbm.at[srco], buf, semg)

        load(0, srci0, srco0, dsti0, buf0, semg0)
        load(1, srci1, srco1, dsti1, buf1, semg1)
        pltpu.make_async_copy(t_hbm.at[srco0], buf0, semg0).wait()
        pltpu.async_copy(buf0, acc.at[dsti0], sems0, add=True)

        def body(k2, carry):
            j = 2 + 2 * k2
            pltpu.make_async_copy(buf0, acc.at[dsti0], sems0).wait()
            load(j, srci0, srco0, dsti0, buf0, semg0)
            pltpu.make_async_copy(t_hbm.at[srco1], buf1, semg1).wait()
            pltpu.async_copy(buf1, acc.at[dsti1], sems1, add=True)
            pltpu.make_async_copy(buf1, acc.at[dsti1], sems1).wait()
            load(j + 1, srci1, srco1, dsti1, buf1, semg1)
            pltpu.make_async_copy(t_hbm.at[srco0], buf0, semg0).wait()
            pltpu.async_copy(buf0, acc.at[dsti0], sems0, add=True)
            return carry

        lax.fori_loop(0, (NB - 2) // 2, body, 0)
        pltpu.make_async_copy(t_hbm.at[srco1], buf1, semg1).wait()
        pltpu.async_copy(buf1, acc.at[dsti1], sems1, add=True)
        pltpu.make_async_copy(buf0, acc.at[dsti0], sems0).wait()
        pltpu.make_async_copy(buf1, acc.at[dsti1], sems1).wait()
        plsc.subcore_barrier()
        pltpu.sync_copy(acc.at[rows], out_hbm.at[c].at[rows])

    return k(table_f, src_p, dst_p, zf)


GB = 64


def _gat_pass(g_f, adst_t, src_p, dst_p, z144):
    NBLK = (EP // 16) // GB

    @functools.partial(
        pl.kernel,
        mesh=_mesh,
        compiler_params=_cp,
        out_type=jax.ShapeDtypeStruct((2, NP, 144), jnp.float32),
        scratch_types=[
            pltpu.VMEM((GB,), jnp.int32),
            pltpu.VMEM((GB,), jnp.int32),
            pltpu.VMEM((GB,), jnp.int32),
            pltpu.VMEM((GB,), jnp.int32),
            pltpu.VMEM((GB,), jnp.int32),
            pltpu.VMEM((GB, 144), jnp.float32),
            pltpu.VMEM((GB, 144), jnp.float32),
            pltpu.VMEM((GB, 16), jnp.float32),
            pltpu.VMEM((GB,), jnp.float32),
            pltpu.VMEM((GB,), jnp.float32),
            pltpu.VMEM_SHARED((NP, 144), jnp.float32),
            pltpu.SemaphoreType.DMA,
            pltpu.SemaphoreType.DMA,
            pltpu.SemaphoreType.DMA,
            pltpu.SemaphoreType.DMA,
        ],
    )
    def k(g_hbm, ad_hbm, s_hbm, d_hbm, z_hbm, out_hbm,
          srci, srco, dsto, dsti0, dsti1, orows0, orows1, adb, ex0, ex1,
          acc, semg, sema, sems0, sems1):
        c = lax.axis_index("c")
        s = lax.axis_index("s")
        rows = pl.ds(s * NPT, NPT)
        pltpu.sync_copy(z_hbm.at[rows], acc.at[rows])
        plsc.subcore_barrier()
        base = s * (EP // 16)
        off = c * NP
        lane = lax.iota(jnp.int32, 16)
        col0 = jnp.zeros((16,), jnp.int32)
        col1 = jnp.full((16,), 1, jnp.int32)
        col2 = jnp.full((16,), 2, jnp.int32)
        col3 = jnp.full((16,), 3, jnp.int32)
        col128 = jnp.full((16,), 128, jnp.int32)
        col129 = jnp.full((16,), 129, jnp.int32)

        def load_compute(j, dsti, orows):
            e0 = base + j * GB
            pltpu.sync_copy(s_hbm.at[pl.ds(e0, GB)], srci)
            pltpu.sync_copy(d_hbm.at[pl.ds(e0, GB)], dsti)
            for g in range(GB // 16):
                sl = pl.ds(g * 16, 16)
                srco[sl] = srci[sl] + off
                dsto[sl] = dsti[sl] + off
            cp0 = pltpu.async_copy(g_hbm.at[srco], orows, semg)
            cp1 = pltpu.async_copy(ad_hbm.at[dsto], adb, sema)
            cp0.wait()
            cp1.wait()
            for g in range(GB // 16):
                sl = pl.ds(g * 16, 16)
                r16 = jnp.full((16,), g * 16, jnp.int32) + lane
                a0 = plsc.load_gather(orows, [r16, col128])
                a1 = plsc.load_gather(orows, [r16, col129])
                b0 = plsc.load_gather(adb, [r16, col0])
                b1 = plsc.load_gather(adb, [r16, col1])
                m0 = plsc.load_gather(adb, [r16, col2])
                m1 = plsc.load_gather(adb, [r16, col3])
                e0v = a0 + b0
                e0v = jnp.where(e0v > 0, e0v, 0.2 * e0v)
                e1v = a1 + b1
                e1v = jnp.where(e1v > 0, e1v, 0.2 * e1v)
                ex0[sl] = jnp.exp(e0v - m0)
                ex1[sl] = jnp.exp(e1v - m1)

            def edge_body(i, carry):
                e0b = plsc.load_gather(ex0, [jnp.full((16,), i, jnp.int32)])
                e1b = plsc.load_gather(ex1, [jnp.full((16,), i, jnp.int32)])
                for f in range(4):
                    sl = pl.ds(f * 16, 16)
                    orows[i, sl] = orows[i, sl] * e0b
                for f in range(4, 8):
                    sl = pl.ds(f * 16, 16)
                    orows[i, sl] = orows[i, sl] * e1b
                tail = jnp.where(lane == 0, e0b,
                                 jnp.where(lane == 1, e1b, jnp.float32(0)))
                orows[i, pl.ds(128, 16)] = tail
                return carry

            lax.fori_loop(0, GB, edge_body, 0)

        load_compute(0, dsti0, orows0)
        pltpu.async_copy(orows0, acc.at[dsti0], sems0, add=True)
        load_compute(1, dsti1, orows1)
        pltpu.async_copy(orows1, acc.at[dsti1], sems1, add=True)

        def body(k2, carry):
            j = 2 + 2 * k2
            pltpu.make_async_copy(orows0, acc.at[dsti0], sems0).wait()
            load_compute(j, dsti0, orows0)
            pltpu.async_copy(orows0, acc.at[dsti0], sems0, add=True)
            pltpu.make_async_copy(orows1, acc.at[dsti1], sems1).wait()
            load_compute(j + 1, dsti1, orows1)
            pltpu.async_copy(orows1, acc.at[dsti1], sems1, add=True)
            return carry

        lax.fori_loop(0, (NBLK - 2) // 2, body, 0)
        pltpu.make_async_copy(orows0, acc.at[dsti0], sems0).wait()
        pltpu.make_async_copy(orows1, acc.at[dsti1], sems1).wait()
        plsc.subcore_barrier()
        pltpu.sync_copy(acc.at[rows], out_hbm.at[c].at[rows])

    return k(g_f, adst_t, src_p, dst_p, z144)


def _tc_a(degp, x_p, W1):
    def body(dp, xr, w1, hs_o, dinv_o):
        deg = dp[0, :, 0:1] + dp[1, :, 0:1]
        dinv = jnp.where(deg > 0, lax.rsqrt(jnp.maximum(deg, 1.0)), 0.0)
        h = jnp.dot(xr[...], w1[...], preferred_element_type=jnp.float32)
        hs_o[...] = h * dinv
        dinv_o[...] = jnp.broadcast_to(dinv, (NP, 8))

    return pl.pallas_call(
        body,
        out_shape=[
            jax.ShapeDtypeStruct((NP, HID), jnp.float32),
            jax.ShapeDtypeStruct((NP, 8), jnp.float32),
        ],
    )(degp, x_p, W1)


def _tc_b(S1, dinv8, b1, Wg, As, Ad):
    def body(s1, dv, b1r, wg, asr, adr, g_o, as_o, ad_o, mp_o):
        S = jnp.concatenate([s1[0], s1[1]], axis=1)
        dinv = dv[:, 0:1]
        h1 = jnp.maximum(S * dinv + b1r[...], 0.0)
        g = jnp.dot(h1, wg[...], preferred_element_type=jnp.float32)
        g_o[...] = g
        a_s = jnp.dot(g, asr[...], preferred_element_type=jnp.float32)
        a_d = jnp.dot(g, adr[...], preferred_element_type=jnp.float32)
        as_o[...] = a_s
        ad_o[...] = a_d
        mg = jnp.max(a_s, axis=0, keepdims=True)
        t = mg + a_d
        mp_o[...] = jnp.where(t > 0, t, 0.2 * t)

    return pl.pallas_call(
        body,
        out_shape=[
            jax.ShapeDtypeStruct((NP, HEADS * HID), jnp.float32),
            jax.ShapeDtypeStruct((NP, HEADS), jnp.float32),
            jax.ShapeDtypeStruct((NP, HEADS), jnp.float32),
            jax.ShapeDtypeStruct((NP, HEADS), jnp.float32),
        ],
    )(S1, dinv8, b1, Wg, As, Ad)


def _ln(h, g, b):
    mu = jnp.mean(h, axis=-1, keepdims=True)
    var = jnp.mean((h - mu) ** 2, axis=-1, keepdims=True)
    return (h - mu) * lax.rsqrt(var + 1e-5) * g + b


def _tc_c(accg, bg, g1, beta1, W2, dinv8):
    def body(ac, bgr, g1r, be1, w2, dv, hs_o):
        pieces = []
        for c in range(2):
            num = ac[c, :, 0:128]
            den0 = ac[c, :, 128:129]
            den1 = ac[c, :, 129:130]
            pieces.append(num[:, 0:64] / (den0 + 1e-16))
            pieces.append(num[:, 64:128] / (den1 + 1e-16))
        gat = jnp.concatenate(pieces, axis=1) + bgr[...]
        h2 = _ln(gat, g1r[...], be1[...])
        hh = jnp.dot(h2, w2[...], preferred_element_type=jnp.float32)
        hs_o[...] = hh * dv[:, 0:1]

    return pl.pallas_call(
        body,
        out_shape=jax.ShapeDtypeStruct((NP, GOUT), jnp.float32),
    )(accg, bg, g1, beta1, W2, dinv8)


def _tc_d(S2, dinv8, b2, g2, beta2, x_p, W3, b3, W4, b4):
    def body(s2, dv, b2r, g2r, be2, xr, w3, b3r, w4, b4r, o):
        S = jnp.concatenate([s2[0], s2[1]], axis=1)
        h3 = S * dv[:, 0:1] + b2r[...]
        h3 = jnp.maximum(_ln(h3, g2r[...], be2[...]), 0.0)
        hc = jnp.concatenate([h3, xr[...]], axis=1)
        h4 = jnp.maximum(
            jnp.dot(hc, w3[...], preferred_element_type=jnp.float32) + b3r[...],
            0.0)
        o[...] = jnp.dot(h4, w4[...], preferred_element_type=jnp.float32) + b4r[...]

    return pl.pallas_call(
        body,
        out_shape=jax.ShapeDtypeStruct((NP, 2), jnp.float32),
    )(S2, dinv8, b2, g2, beta2, x_p, W3, b3, W4, b4)


def kernel(x, edge_index, W1, b1, Wg, att_src, att_dst, bg, g1, beta1,
           W2, b2, g2, beta2, W3, b3, W4, b4):
    loop = jnp.arange(N, dtype=jnp.int32)
    src = jnp.concatenate([edge_index[0], loop])
    dst = jnp.concatenate([edge_index[1], loop])
    pad = jnp.full((EP - ET,), NP - 1, jnp.int32)
    src_p = jnp.concatenate([src, pad])
    dst_p = jnp.concatenate([dst, pad])
    x_p = jnp.pad(x, ((0, NP - N), (0, 0)))

    val8 = jnp.concatenate(
        [jnp.ones((EB, 1), jnp.float32), jnp.zeros((EB, 7), jnp.float32)], axis=1)
    z8 = jnp.zeros((NP, 8), jnp.float32)
    z32 = jnp.zeros((NP, 32), jnp.float32)
    z64 = jnp.zeros((NP, 64), jnp.float32)
    z144 = jnp.zeros((NP, 144), jnp.float32)

    degp = _deg_call(dst_p, val8, z8)
    hs1, dinv8 = _tc_a(degp, x_p, W1)
    hs1f = jnp.concatenate([hs1[:, :32], hs1[:, 32:]], axis=0)
    S1 = _gcn_pass(hs1f, src_p, dst_p, z32, 32)

    head = jnp.arange(HEADS * HID, dtype=jnp.int32) // HID
    sel = (head[:, None] == jnp.arange(HEADS, dtype=jnp.int32)[None, :])
    As = jnp.where(sel, att_src.reshape(-1)[:, None], 0.0)
    Ad = jnp.where(sel, att_dst.reshape(-1)[:, None], 0.0)

    g, a_s, a_d, mp = _tc_b(S1, dinv8, b1, Wg, As, Ad)
    gf = jnp.concatenate([
        jnp.pad(jnp.concatenate([g[:, :128], a_s[:, 0:2]], axis=1),
                ((0, 0), (0, 14))),
        jnp.pad(jnp.concatenate([g[:, 128:], a_s[:, 2:4]], axis=1),
                ((0, 0), (0, 14))),
    ], axis=0)
    adst_t = jnp.concatenate([
        jnp.pad(jnp.concatenate([a_d[:, 0:2], mp[:, 0:2]], axis=1),
                ((0, 0), (0, 12))),
        jnp.pad(jnp.concatenate([a_d[:, 2:4], mp[:, 2:4]], axis=1),
                ((0, 0), (0, 12))),
    ], axis=0)
    accg = _gat_pass(gf, adst_t, src_p, dst_p, z144)

    hs2 = _tc_c(accg, bg, g1, beta1, W2, dinv8)
    hs2f = jnp.concatenate([hs2[:, :64], hs2[:, 64:]], axis=0)
    S2 = _gcn_pass(hs2f, src_p, dst_p, z64, 64)

    out = _tc_d(S2, dinv8, b2, g2, beta2, x_p, W3, b3, W4, b4)
    return out[:N]


# speedup vs baseline: 24.6889x; 1.2415x over previous
import functools

import jax
import jax.numpy as jnp
from jax import lax
from jax.experimental import pallas as pl
from jax.experimental.pallas import tpu as pltpu
from jax.experimental.pallas import tpu_sc as plsc

N = 10000
NP = 10240
NPT = NP // 16
DIN = 128
HID = 64
HEADS = 4
GOUT = 128

ET = 330000
EP = 344064
EB = 128

_mesh = plsc.VectorSubcoreMesh(core_axis_name="c", subcore_axis_name="s")
_cp = pltpu.CompilerParams(use_tc_tiling_on_sc=False, needs_layout_passes=False)


def _add_offset(idx_ref, out_ref, off):
    for g in range(EB // 16):
        sl = pl.ds(g * 16, 16)
        out_ref[sl] = idx_ref[sl] + off


def _deg_call(dst_p, val8, z8):
    @functools.partial(
        pl.kernel,
        mesh=_mesh,
        compiler_params=_cp,
        out_type=jax.ShapeDtypeStruct((2, NP, 8), jnp.float32),
        scratch_types=[
            pltpu.VMEM((EB,), jnp.int32),
            pltpu.VMEM((EB, 8), jnp.float32),
            pltpu.VMEM_SHARED((NP, 8), jnp.float32),
        ],
    )
    def k(dst_hbm, val_hbm, z_hbm, out_hbm, dsti, val_v, acc):
        c = lax.axis_index("c")
        s = lax.axis_index("s")
        rows = pl.ds(s * NPT, NPT)
        pltpu.sync_copy(z_hbm.at[rows], acc.at[rows])
        pltpu.sync_copy(val_hbm, val_v)
        plsc.subcore_barrier()
        base = c * (EP // 2) + s * (EP // 32)

        def body(j, carry):
            pltpu.sync_copy(dst_hbm.at[pl.ds(base + j * EB, EB)], dsti)
            pltpu.sync_copy(val_v, acc.at[dsti], add=True)
            return carry

        lax.fori_loop(0, (EP // 32) // EB, body, 0)
        plsc.subcore_barrier()
        pltpu.sync_copy(acc.at[rows], out_hbm.at[c].at[rows])

    return k(dst_p, val8, z8)


def _gcn_pass(table_f, src_p, dst_p, zf, fh):
    NB = (EP // 16) // EB

    @functools.partial(
        pl.kernel,
        mesh=_mesh,
        compiler_params=_cp,
        out_type=jax.ShapeDtypeStruct((2, NP, fh), jnp.float32),
        scratch_types=[
            pltpu.VMEM((EB,), jnp.int32),
            pltpu.VMEM((EB,), jnp.int32),
            pltpu.VMEM((EB,), jnp.int32),
            pltpu.VMEM((EB, fh), jnp.float32),
            pltpu.VMEM((EB,), jnp.int32),
            pltpu.VMEM((EB,), jnp.int32),
            pltpu.VMEM((EB,), jnp.int32),
            pltpu.VMEM((EB, fh), jnp.float32),
            pltpu.VMEM_SHARED((NP, fh), jnp.float32),
            pltpu.SemaphoreType.DMA,
            pltpu.SemaphoreType.DMA,
            pltpu.SemaphoreType.DMA,
            pltpu.SemaphoreType.DMA,
        ],
    )
    def k(t_hbm, s_hbm, d_hbm, z_hbm, out_hbm,
          srci0, srco0, dsti0, buf0, srci1, srco1, dsti1, buf1,
          acc, semg0, semg1, sems0, sems1):
        c = lax.axis_index("c")
        s = lax.axis_index("s")
        rows = pl.ds(s * NPT, NPT)
        pltpu.sync_copy(z_hbm.at[rows], acc.at[rows])
        plsc.subcore_barrier()
        base = s * (EP // 16)
        off = c * NP

        def load(j, srci, srco, dsti, buf, semg):
            e0 = base + j * EB
            pltpu.sync_copy(s_hbm.at[pl.ds(e0, EB)], srci)
            pltpu.sync_copy(d_hbm.at[pl.ds(e0, EB)], dsti)
            _add_offset(srci, srco, off)
            pltpu.async_copy(t_hbm.at[srco], buf, semg)

        load(0, srci0, srco0, dsti0, buf0, semg0)
        load(1, srci1, srco1, dsti1, buf1, semg1)
        pltpu.make_async_copy(t_hbm.at[srco0], buf0, semg0).wait()
        pltpu.async_copy(buf0, acc.at[dsti0], sems0, add=True)

        def body(k2, carry):
            j = 2 + 2 * k2
            pltpu.make_async_copy(buf0, acc.at[dsti0], sems0).wait()
            load(j, srci0, srco0, dsti0, buf0, semg0)
            pltpu.make_async_copy(t_hbm.at[srco1], buf1, semg1).wait()
            pltpu.async_copy(buf1, acc.at[dsti1], sems1, add=True)
            pltpu.make_async_copy(buf1, acc.at[dsti1], sems1).wait()
            load(j + 1, srci1, srco1, dsti1, buf1, semg1)
            pltpu.make_async_copy(t_hbm.at[srco0], buf0, semg0).wait()
            pltpu.async_copy(buf0, acc.at[dsti0], sems0, add=True)
            return carry

        lax.fori_loop(0, (NB - 2) // 2, body, 0)
        pltpu.make_async_copy(t_hbm.at[srco1], buf1, semg1).wait()
        pltpu.async_copy(buf1, acc.at[dsti1], sems1, add=True)
        pltpu.make_async_copy(buf0, acc.at[dsti0], sems0).wait()
        pltpu.make_async_copy(buf1, acc.at[dsti1], sems1).wait()
        plsc.subcore_barrier()
        pltpu.sync_copy(acc.at[rows], out_hbm.at[c].at[rows])

    return k(table_f, src_p, dst_p, zf)


GB = 64


def _gat_pass(g_f, adst_t, src_p, dst_p, z144):
    NBLK = (EP // 16) // GB

    @functools.partial(
        pl.kernel,
        mesh=_mesh,
        compiler_params=_cp,
        out_type=jax.ShapeDtypeStruct((2, NP, 144), jnp.float32),
        scratch_types=[
            pltpu.VMEM((GB,), jnp.int32),
            pltpu.VMEM((GB,), jnp.int32),
            pltpu.VMEM((GB,), jnp.int32),
            pltpu.VMEM((GB,), jnp.int32),
            pltpu.VMEM((GB, 144), jnp.float32),
            pltpu.VMEM((GB, 16), jnp.float32),
            pltpu.VMEM((GB,), jnp.float32),
            pltpu.VMEM((GB,), jnp.float32),
            pltpu.VMEM((GB,), jnp.int32),
            pltpu.VMEM((GB,), jnp.int32),
            pltpu.VMEM((GB,), jnp.int32),
            pltpu.VMEM((GB,), jnp.int32),
            pltpu.VMEM((GB, 144), jnp.float32),
            pltpu.VMEM((GB, 16), jnp.float32),
            pltpu.VMEM((GB,), jnp.float32),
            pltpu.VMEM((GB,), jnp.float32),
            pltpu.VMEM_SHARED((NP, 144), jnp.float32),
            pltpu.SemaphoreType.DMA,
            pltpu.SemaphoreType.DMA,
            pltpu.SemaphoreType.DMA,
            pltpu.SemaphoreType.DMA,
            pltpu.SemaphoreType.DMA,
            pltpu.SemaphoreType.DMA,
        ],
    )
    def k(g_hbm, ad_hbm, s_hbm, d_hbm, z_hbm, out_hbm,
          srci0, srco0, dsto0, dsti0, orows0, adb0, ex00, ex10,
          srci1, srco1, dsto1, dsti1, orows1, adb1, ex01, ex11,
          acc, semg0, sema0, sems0, semg1, sema1, sems1):
        c = lax.axis_index("c")
        s = lax.axis_index("s")
        rows = pl.ds(s * NPT, NPT)
        pltpu.sync_copy(z_hbm.at[rows], acc.at[rows])
        plsc.subcore_barrier()
        base = s * (EP // 16)
        off = c * NP
        lane = lax.iota(jnp.int32, 16)
        col0 = jnp.zeros((16,), jnp.int32)
        col1 = jnp.full((16,), 1, jnp.int32)
        col2 = jnp.full((16,), 2, jnp.int32)
        col3 = jnp.full((16,), 3, jnp.int32)
        col128 = jnp.full((16,), 128, jnp.int32)
        col129 = jnp.full((16,), 129, jnp.int32)

        def load(j, srci, srco, dsto, dsti, orows, adb, semg, sema):
            e0 = base + j * GB
            pltpu.sync_copy(s_hbm.at[pl.ds(e0, GB)], srci)
            pltpu.sync_copy(d_hbm.at[pl.ds(e0, GB)], dsti)
            for g in range(GB // 16):
                sl = pl.ds(g * 16, 16)
                srco[sl] = srci[sl] + off
                dsto[sl] = dsti[sl] + off
            pltpu.async_copy(g_hbm.at[srco], orows, semg)
            pltpu.async_copy(ad_hbm.at[dsto], adb, sema)

        def comp(srco, dsto, dsti, orows, adb, ex0, ex1, semg, sema, sems):
            pltpu.make_async_copy(g_hbm.at[srco], orows, semg).wait()
            pltpu.make_async_copy(ad_hbm.at[dsto], adb, sema).wait()
            for g in range(GB // 16):
                sl = pl.ds(g * 16, 16)
                r16 = jnp.full((16,), g * 16, jnp.int32) + lane
                a0 = plsc.load_gather(orows, [r16, col128])
                a1 = plsc.load_gather(orows, [r16, col129])
                b0 = plsc.load_gather(adb, [r16, col0])
                b1 = plsc.load_gather(adb, [r16, col1])
                m0 = plsc.load_gather(adb, [r16, col2])
                m1 = plsc.load_gather(adb, [r16, col3])
                e0v = a0 + b0
                e0v = jnp.where(e0v > 0, e0v, 0.2 * e0v)
                e1v = a1 + b1
                e1v = jnp.where(e1v > 0, e1v, 0.2 * e1v)
                ex0[sl] = jnp.exp(e0v - m0)
                ex1[sl] = jnp.exp(e1v - m1)

            def edge_body(i, carry):
                e0b = plsc.load_gather(ex0, [jnp.full((16,), i, jnp.int32)])
                e1b = plsc.load_gather(ex1, [jnp.full((16,), i, jnp.int32)])
                for f in range(4):
                    sl = pl.ds(f * 16, 16)
                    orows[i, sl] = orows[i, sl] * e0b
                for f in range(4, 8):
                    sl = pl.ds(f * 16, 16)
                    orows[i, sl] = orows[i, sl] * e1b
                tail = jnp.where(lane == 0, e0b,
                                 jnp.where(lane == 1, e1b, jnp.float32(0)))
                orows[i, pl.ds(128, 16)] = tail
                return carry

            lax.fori_loop(0, GB, edge_body, 0)
            pltpu.async_copy(orows, acc.at[dsti], sems, add=True)

        load(0, srci0, srco0, dsto0, dsti0, orows0, adb0, semg0, sema0)
        load(1, srci1, srco1, dsto1, dsti1, orows1, adb1, semg1, sema1)
        comp(srco0, dsto0, dsti0, orows0, adb0, ex00, ex10,
             semg0, sema0, sems0)

        def body(k2, carry):
            j = 2 + 2 * k2
            pltpu.make_async_copy(orows0, acc.at[dsti0], sems0).wait()
            load(j, srci0, srco0, dsto0, dsti0, orows0, adb0, semg0, sema0)
            comp(srco1, dsto1, dsti1, orows1, adb1, ex01, ex11,
                 semg1, sema1, sems1)
            pltpu.make_async_copy(orows1, acc.at[dsti1], sems1).wait()
            load(j + 1, srci1, srco1, dsto1, dsti1, orows1, adb1, semg1, sema1)
            comp(srco0, dsto0, dsti0, orows0, adb0, ex00, ex10,
                 semg0, sema0, sems0)
            return carry

        lax.fori_loop(0, (NBLK - 2) // 2, body, 0)
        comp(srco1, dsto1, dsti1, orows1, adb1, ex01, ex11,
             semg1, sema1, sems1)
        pltpu.make_async_copy(orows0, acc.at[dsti0], sems0).wait()
        pltpu.make_async_copy(orows1, acc.at[dsti1], sems1).wait()
        plsc.subcore_barrier()
        pltpu.sync_copy(acc.at[rows], out_hbm.at[c].at[rows])

    return k(g_f, adst_t, src_p, dst_p, z144)


def _tc_a(degp, x_p, W1):
    def body(dp, xr, w1, hs_o, dinv_o):
        deg = dp[0, :, 0:1] + dp[1, :, 0:1]
        dinv = jnp.where(deg > 0, lax.rsqrt(jnp.maximum(deg, 1.0)), 0.0)
        h = jnp.dot(xr[...], w1[...], preferred_element_type=jnp.float32)
        hs_o[...] = h * dinv
        dinv_o[...] = jnp.broadcast_to(dinv, (NP, 8))

    return pl.pallas_call(
        body,
        out_shape=[
            jax.ShapeDtypeStruct((NP, HID), jnp.float32),
            jax.ShapeDtypeStruct((NP, 8), jnp.float32),
        ],
    )(degp, x_p, W1)


def _tc_b(S1, dinv8, b1, Wg, As, Ad):
    def body(s1, dv, b1r, wg, asr, adr, g_o, as_o, ad_o, mp_o):
        S = jnp.concatenate([s1[0], s1[1]], axis=1)
        dinv = dv[:, 0:1]
        h1 = jnp.maximum(S * dinv + b1r[...], 0.0)
        g = jnp.dot(h1, wg[...], preferred_element_type=jnp.float32)
        g_o[...] = g
        a_s = jnp.dot(g, asr[...], preferred_element_type=jnp.float32)
        a_d = jnp.dot(g, adr[...], preferred_element_type=jnp.float32)
        as_o[...] = a_s
        ad_o[...] = a_d
        mg = jnp.max(a_s, axis=0, keepdims=True)
        t = mg + a_d
        mp_o[...] = jnp.where(t > 0, t, 0.2 * t)

    return pl.pallas_call(
        body,
        out_shape=[
            jax.ShapeDtypeStruct((NP, HEADS * HID), jnp.float32),
            jax.ShapeDtypeStruct((NP, HEADS), jnp.float32),
            jax.ShapeDtypeStruct((NP, HEADS), jnp.float32),
            jax.ShapeDtypeStruct((NP, HEADS), jnp.float32),
        ],
    )(S1, dinv8, b1, Wg, As, Ad)


def _ln(h, g, b):
    mu = jnp.mean(h, axis=-1, keepdims=True)
    var = jnp.mean((h - mu) ** 2, axis=-1, keepdims=True)
    return (h - mu) * lax.rsqrt(var + 1e-5) * g + b


def _tc_c(accg, bg, g1, beta1, W2, dinv8):
    def body(ac, bgr, g1r, be1, w2, dv, hs_o):
        pieces = []
        for c in range(2):
            num = ac[c, :, 0:128]
            den0 = ac[c, :, 128:129]
            den1 = ac[c, :, 129:130]
            pieces.append(num[:, 0:64] / (den0 + 1e-16))
            pieces.append(num[:, 64:128] / (den1 + 1e-16))
        gat = jnp.concatenate(pieces, axis=1) + bgr[...]
        h2 = _ln(gat, g1r[...], be1[...])
        hh = jnp.dot(h2, w2[...], preferred_element_type=jnp.float32)
        hs_o[...] = hh * dv[:, 0:1]

    return pl.pallas_call(
        body,
        out_shape=jax.ShapeDtypeStruct((NP, GOUT), jnp.float32),
    )(accg, bg, g1, beta1, W2, dinv8)


def _tc_d(S2, dinv8, b2, g2, beta2, x_p, W3, b3, W4, b4):
    def body(s2, dv, b2r, g2r, be2, xr, w3, b3r, w4, b4r, o):
        S = jnp.concatenate([s2[0], s2[1]], axis=1)
        h3 = S * dv[:, 0:1] + b2r[...]
        h3 = jnp.maximum(_ln(h3, g2r[...], be2[...]), 0.0)
        hc = jnp.concatenate([h3, xr[...]], axis=1)
        h4 = jnp.maximum(
            jnp.dot(hc, w3[...], preferred_element_type=jnp.float32) + b3r[...],
            0.0)
        o[...] = jnp.dot(h4, w4[...], preferred_element_type=jnp.float32) + b4r[...]

    return pl.pallas_call(
        body,
        out_shape=jax.ShapeDtypeStruct((NP, 2), jnp.float32),
    )(S2, dinv8, b2, g2, beta2, x_p, W3, b3, W4, b4)


def kernel(x, edge_index, W1, b1, Wg, att_src, att_dst, bg, g1, beta1,
           W2, b2, g2, beta2, W3, b3, W4, b4):
    loop = jnp.arange(N, dtype=jnp.int32)
    src = jnp.concatenate([edge_index[0], loop])
    dst = jnp.concatenate([edge_index[1], loop])
    pad = jnp.full((EP - ET,), NP - 1, jnp.int32)
    src_p = jnp.concatenate([src, pad])
    dst_p = jnp.concatenate([dst, pad])
    x_p = jnp.pad(x, ((0, NP - N), (0, 0)))

    val8 = jnp.concatenate(
        [jnp.ones((EB, 1), jnp.float32), jnp.zeros((EB, 7), jnp.float32)], axis=1)
    z8 = jnp.zeros((NP, 8), jnp.float32)
    z32 = jnp.zeros((NP, 32), jnp.float32)
    z64 = jnp.zeros((NP, 64), jnp.float32)
    z144 = jnp.zeros((NP, 144), jnp.float32)

    degp = _deg_call(dst_p, val8, z8)
    hs1, dinv8 = _tc_a(degp, x_p, W1)
    hs1f = jnp.concatenate([hs1[:, :32], hs1[:, 32:]], axis=0)
    S1 = _gcn_pass(hs1f, src_p, dst_p, z32, 32)

    head = jnp.arange(HEADS * HID, dtype=jnp.int32) // HID
    sel = (head[:, None] == jnp.arange(HEADS, dtype=jnp.int32)[None, :])
    As = jnp.where(sel, att_src.reshape(-1)[:, None], 0.0)
    Ad = jnp.where(sel, att_dst.reshape(-1)[:, None], 0.0)

    g, a_s, a_d, mp = _tc_b(S1, dinv8, b1, Wg, As, Ad)
    gf = jnp.concatenate([
        jnp.pad(jnp.concatenate([g[:, :128], a_s[:, 0:2]], axis=1),
                ((0, 0), (0, 14))),
        jnp.pad(jnp.concatenate([g[:, 128:], a_s[:, 2:4]], axis=1),
                ((0, 0), (0, 14))),
    ], axis=0)
    adst_t = jnp.concatenate([
        jnp.pad(jnp.concatenate([a_d[:, 0:2], mp[:, 0:2]], axis=1),
                ((0, 0), (0, 12))),
        jnp.pad(jnp.concatenate([a_d[:, 2:4], mp[:, 2:4]], axis=1),
                ((0, 0), (0, 12))),
    ], axis=0)
    accg = _gat_pass(gf, adst_t, src_p, dst_p, z144)

    hs2 = _tc_c(accg, bg, g1, beta1, W2, dinv8)
    hs2f = jnp.concatenate([hs2[:, :64], hs2[:, 64:]], axis=0)
    S2 = _gcn_pass(hs2f, src_p, dst_p, z64, 64)

    out = _tc_d(S2, dinv8, b2, g2, beta2, x_p, W3, b3, W4, b4)
    return out[:N]


# speedup vs baseline: 24.9749x; 1.0116x over previous
import functools

import jax
import jax.numpy as jnp
from jax import lax
from jax.experimental import pallas as pl
from jax.experimental.pallas import tpu as pltpu
from jax.experimental.pallas import tpu_sc as plsc

N = 10000
NP = 10240
NPT = NP // 16
DIN = 128
HID = 64
HEADS = 4
GOUT = 128

ET = 330000
EP = 344064
EB = 128

_mesh = plsc.VectorSubcoreMesh(core_axis_name="c", subcore_axis_name="s")
_cp = pltpu.CompilerParams(use_tc_tiling_on_sc=False, needs_layout_passes=False)


def _add_offset(idx_ref, out_ref, off):
    for g in range(EB // 16):
        sl = pl.ds(g * 16, 16)
        out_ref[sl] = idx_ref[sl] + off


def _deg_call(dst_p, val8, z8):
    @functools.partial(
        pl.kernel,
        mesh=_mesh,
        compiler_params=_cp,
        out_type=jax.ShapeDtypeStruct((2, NP, 8), jnp.float32),
        scratch_types=[
            pltpu.VMEM((EB,), jnp.int32),
            pltpu.VMEM((EB, 8), jnp.float32),
            pltpu.VMEM_SHARED((NP, 8), jnp.float32),
        ],
    )
    def k(dst_hbm, val_hbm, z_hbm, out_hbm, dsti, val_v, acc):
        c = lax.axis_index("c")
        s = lax.axis_index("s")
        rows = pl.ds(s * NPT, NPT)
        pltpu.sync_copy(z_hbm.at[rows], acc.at[rows])
        pltpu.sync_copy(val_hbm, val_v)
        plsc.subcore_barrier()
        base = c * (EP // 2) + s * (EP // 32)

        def body(j, carry):
            pltpu.sync_copy(dst_hbm.at[pl.ds(base + j * EB, EB)], dsti)
            pltpu.sync_copy(val_v, acc.at[dsti], add=True)
            return carry

        lax.fori_loop(0, (EP // 32) // EB, body, 0)
        plsc.subcore_barrier()
        pltpu.sync_copy(acc.at[rows], out_hbm.at[c].at[rows])

    return k(dst_p, val8, z8)


def _gcn_pass(table_f, src_p, dst_p, zf, fh):
    NB = (EP // 16) // EB

    @functools.partial(
        pl.kernel,
        mesh=_mesh,
        compiler_params=_cp,
        out_type=jax.ShapeDtypeStruct((2, NP, fh), jnp.float32),
        scratch_types=[
            pltpu.VMEM((EB,), jnp.int32),
            pltpu.VMEM((EB,), jnp.int32),
            pltpu.VMEM((EB,), jnp.int32),
            pltpu.VMEM((EB, fh), jnp.float32),
            pltpu.VMEM((EB,), jnp.int32),
            pltpu.VMEM((EB,), jnp.int32),
            pltpu.VMEM((EB,), jnp.int32),
            pltpu.VMEM((EB, fh), jnp.float32),
            pltpu.VMEM_SHARED((NP, fh), jnp.float32),
            pltpu.SemaphoreType.DMA,
            pltpu.SemaphoreType.DMA,
            pltpu.SemaphoreType.DMA,
            pltpu.SemaphoreType.DMA,
        ],
    )
    def k(t_hbm, s_hbm, d_hbm, z_hbm, out_hbm,
          srci0, srco0, dsti0, buf0, srci1, srco1, dsti1, buf1,
          acc, semg0, semg1, sems0, sems1):
        c = lax.axis_index("c")
        s = lax.axis_index("s")
        rows = pl.ds(s * NPT, NPT)
        pltpu.sync_copy(z_hbm.at[rows], acc.at[rows])
        plsc.subcore_barrier()
        base = s * (EP // 16)
        off = c * NP

        def load(j, srci, srco, dsti, buf, semg):
            e0 = base + j * EB
            pltpu.sync_copy(s_hbm.at[pl.ds(e0, EB)], srci)
            pltpu.sync_copy(d_hbm.at[pl.ds(e0, EB)], dsti)
            _add_offset(srci, srco, off)
            pltpu.async_copy(t_hbm.at[srco], buf, semg)

        load(0, srci0, srco0, dsti0, buf0, semg0)
        load(1, srci1, srco1, dsti1, buf1, semg1)
        pltpu.make_async_copy(t_hbm.at[srco0], buf0, semg0).wait()
        pltpu.async_copy(buf0, acc.at[dsti0], sems0, add=True)

        def body(k2, carry):
            j = 2 + 2 * k2
            pltpu.make_async_copy(buf0, acc.at[dsti0], sems0).wait()
            load(j, srci0, srco0, dsti0, buf0, semg0)
            pltpu.make_async_copy(t_hbm.at[srco1], buf1, semg1).wait()
            pltpu.async_copy(buf1, acc.at[dsti1], sems1, add=True)
            pltpu.make_async_copy(buf1, acc.at[dsti1], sems1).wait()
            load(j + 1, srci1, srco1, dsti1, buf1, semg1)
            pltpu.make_async_copy(t_hbm.at[srco0], buf0, semg0).wait()
            pltpu.async_copy(buf0, acc.at[dsti0], sems0, add=True)
            return carry

        lax.fori_loop(0, (NB - 2) // 2, body, 0)
        pltpu.make_async_copy(t_hbm.at[srco1], buf1, semg1).wait()
        pltpu.async_copy(buf1, acc.at[dsti1], sems1, add=True)
        pltpu.make_async_copy(buf0, acc.at[dsti0], sems0).wait()
        pltpu.make_async_copy(buf1, acc.at[dsti1], sems1).wait()
        plsc.subcore_barrier()
        pltpu.sync_copy(acc.at[rows], out_hbm.at[c].at[rows])

    return k(table_f, src_p, dst_p, zf)


GB = 96


def _gat_pass(g_f, adst_t, src_p, dst_p, z144):
    NBLK = (EP // 16) // GB

    @functools.partial(
        pl.kernel,
        mesh=_mesh,
        compiler_params=_cp,
        out_type=jax.ShapeDtypeStruct((2, NP, 144), jnp.float32),
        scratch_types=[
            pltpu.VMEM((GB,), jnp.int32),
            pltpu.VMEM((GB,), jnp.int32),
            pltpu.VMEM((GB,), jnp.int32),
            pltpu.VMEM((GB,), jnp.int32),
            pltpu.VMEM((GB, 144), jnp.float32),
            pltpu.VMEM((GB, 16), jnp.float32),
            pltpu.VMEM((GB,), jnp.float32),
            pltpu.VMEM((GB,), jnp.float32),
            pltpu.VMEM((GB,), jnp.int32),
            pltpu.VMEM((GB,), jnp.int32),
            pltpu.VMEM((GB,), jnp.int32),
            pltpu.VMEM((GB,), jnp.int32),
            pltpu.VMEM((GB, 144), jnp.float32),
            pltpu.VMEM((GB, 16), jnp.float32),
            pltpu.VMEM((GB,), jnp.float32),
            pltpu.VMEM((GB,), jnp.float32),
            pltpu.VMEM_SHARED((NP, 144), jnp.float32),
            pltpu.SemaphoreType.DMA,
            pltpu.SemaphoreType.DMA,
            pltpu.SemaphoreType.DMA,
            pltpu.SemaphoreType.DMA,
            pltpu.SemaphoreType.DMA,
            pltpu.SemaphoreType.DMA,
        ],
    )
    def k(g_hbm, ad_hbm, s_hbm, d_hbm, z_hbm, out_hbm,
          srci0, srco0, dsto0, dsti0, orows0, adb0, ex00, ex10,
          srci1, srco1, dsto1, dsti1, orows1, adb1, ex01, ex11,
          acc, semg0, sema0, sems0, semg1, sema1, sems1):
        c = lax.axis_index("c")
        s = lax.axis_index("s")
        rows = pl.ds(s * NPT, NPT)
        pltpu.sync_copy(z_hbm.at[rows], acc.at[rows])
        plsc.subcore_barrier()
        base = s * (EP // 16)
        off = c * NP
        lane = lax.iota(jnp.int32, 16)
        col0 = jnp.zeros((16,), jnp.int32)
        col1 = jnp.full((16,), 1, jnp.int32)
        col2 = jnp.full((16,), 2, jnp.int32)
        col3 = jnp.full((16,), 3, jnp.int32)
        col128 = jnp.full((16,), 128, jnp.int32)
        col129 = jnp.full((16,), 129, jnp.int32)

        def load(j, srci, srco, dsto, dsti, orows, adb, semg, sema):
            e0 = base + j * GB
            pltpu.sync_copy(s_hbm.at[pl.ds(e0, GB)], srci)
            pltpu.sync_copy(d_hbm.at[pl.ds(e0, GB)], dsti)
            for g in range(GB // 16):
                sl = pl.ds(g * 16, 16)
                srco[sl] = srci[sl] + off
                dsto[sl] = dsti[sl] + off
            pltpu.async_copy(g_hbm.at[srco], orows, semg)
            pltpu.async_copy(ad_hbm.at[dsto], adb, sema)

        def comp(srco, dsto, dsti, orows, adb, ex0, ex1, semg, sema, sems):
            pltpu.make_async_copy(g_hbm.at[srco], orows, semg).wait()
            pltpu.make_async_copy(ad_hbm.at[dsto], adb, sema).wait()
            for g in range(GB // 16):
                sl = pl.ds(g * 16, 16)
                r16 = jnp.full((16,), g * 16, jnp.int32) + lane
                a0 = plsc.load_gather(orows, [r16, col128])
                a1 = plsc.load_gather(orows, [r16, col129])
                b0 = plsc.load_gather(adb, [r16, col0])
                b1 = plsc.load_gather(adb, [r16, col1])
                m0 = plsc.load_gather(adb, [r16, col2])
                m1 = plsc.load_gather(adb, [r16, col3])
                e0v = a0 + b0
                e0v = jnp.where(e0v > 0, e0v, 0.2 * e0v)
                e1v = a1 + b1
                e1v = jnp.where(e1v > 0, e1v, 0.2 * e1v)
                ex0[sl] = jnp.exp(e0v - m0)
                ex1[sl] = jnp.exp(e1v - m1)

            def edge_body(i, carry):
                e0b = plsc.load_gather(ex0, [jnp.full((16,), i, jnp.int32)])
                e1b = plsc.load_gather(ex1, [jnp.full((16,), i, jnp.int32)])
                for f in range(4):
                    sl = pl.ds(f * 16, 16)
                    orows[i, sl] = orows[i, sl] * e0b
                for f in range(4, 8):
                    sl = pl.ds(f * 16, 16)
                    orows[i, sl] = orows[i, sl] * e1b
                tail = jnp.where(lane == 0, e0b,
                                 jnp.where(lane == 1, e1b, jnp.float32(0)))
                orows[i, pl.ds(128, 16)] = tail
                return carry

            lax.fori_loop(0, GB, edge_body, 0)
            pltpu.async_copy(orows, acc.at[dsti], sems, add=True)

        load(0, srci0, srco0, dsto0, dsti0, orows0, adb0, semg0, sema0)
        load(1, srci1, srco1, dsto1, dsti1, orows1, adb1, semg1, sema1)
        comp(srco0, dsto0, dsti0, orows0, adb0, ex00, ex10,
             semg0, sema0, sems0)

        def body(k2, carry):
            j = 2 + 2 * k2
            pltpu.make_async_copy(orows0, acc.at[dsti0], sems0).wait()
            load(j, srci0, srco0, dsto0, dsti0, orows0, adb0, semg0, sema0)
            comp(srco1, dsto1, dsti1, orows1, adb1, ex01, ex11,
                 semg1, sema1, sems1)
            pltpu.make_async_copy(orows1, acc.at[dsti1], sems1).wait()
            load(j + 1, srci1, srco1, dsto1, dsti1, orows1, adb1, semg1, sema1)
            comp(srco0, dsto0, dsti0, orows0, adb0, ex00, ex10,
                 semg0, sema0, sems0)
            return carry

        lax.fori_loop(0, (NBLK - 2) // 2, body, 0)
        comp(srco1, dsto1, dsti1, orows1, adb1, ex01, ex11,
             semg1, sema1, sems1)
        pltpu.make_async_copy(orows0, acc.at[dsti0], sems0).wait()
        pltpu.make_async_copy(orows1, acc.at[dsti1], sems1).wait()
        plsc.subcore_barrier()
        pltpu.sync_copy(acc.at[rows], out_hbm.at[c].at[rows])

    return k(g_f, adst_t, src_p, dst_p, z144)


def _tc_a(degp, x_p, W1):
    def body(dp, xr, w1, hs_o, dinv_o):
        deg = dp[0, :, 0:1] + dp[1, :, 0:1]
        dinv = jnp.where(deg > 0, lax.rsqrt(jnp.maximum(deg, 1.0)), 0.0)
        h = jnp.dot(xr[...], w1[...], preferred_element_type=jnp.float32)
        hs_o[...] = h * dinv
        dinv_o[...] = jnp.broadcast_to(dinv, (NP, 8))

    return pl.pallas_call(
        body,
        out_shape=[
            jax.ShapeDtypeStruct((NP, HID), jnp.float32),
            jax.ShapeDtypeStruct((NP, 8), jnp.float32),
        ],
    )(degp, x_p, W1)


def _tc_b(S1, dinv8, b1, Wg, As, Ad):
    def body(s1, dv, b1r, wg, asr, adr, g_o, as_o, ad_o, mp_o):
        S = jnp.concatenate([s1[0], s1[1]], axis=1)
        dinv = dv[:, 0:1]
        h1 = jnp.maximum(S * dinv + b1r[...], 0.0)
        g = jnp.dot(h1, wg[...], preferred_element_type=jnp.float32)
        g_o[...] = g
        a_s = jnp.dot(g, asr[...], preferred_element_type=jnp.float32)
        a_d = jnp.dot(g, adr[...], preferred_element_type=jnp.float32)
        as_o[...] = a_s
        ad_o[...] = a_d
        mg = jnp.max(a_s, axis=0, keepdims=True)
        t = mg + a_d
        mp_o[...] = jnp.where(t > 0, t, 0.2 * t)

    return pl.pallas_call(
        body,
        out_shape=[
            jax.ShapeDtypeStruct((NP, HEADS * HID), jnp.float32),
            jax.ShapeDtypeStruct((NP, HEADS), jnp.float32),
            jax.ShapeDtypeStruct((NP, HEADS), jnp.float32),
            jax.ShapeDtypeStruct((NP, HEADS), jnp.float32),
        ],
    )(S1, dinv8, b1, Wg, As, Ad)


def _ln(h, g, b):
    mu = jnp.mean(h, axis=-1, keepdims=True)
    var = jnp.mean((h - mu) ** 2, axis=-1, keepdims=True)
    return (h - mu) * lax.rsqrt(var + 1e-5) * g + b


def _tc_c(accg, bg, g1, beta1, W2, dinv8):
    def body(ac, bgr, g1r, be1, w2, dv, hs_o):
        pieces = []
        for c in range(2):
            num = ac[c, :, 0:128]
            den0 = ac[c, :, 128:129]
            den1 = ac[c, :, 129:130]
            pieces.append(num[:, 0:64] / (den0 + 1e-16))
            pieces.append(num[:, 64:128] / (den1 + 1e-16))
        gat = jnp.concatenate(pieces, axis=1) + bgr[...]
        h2 = _ln(gat, g1r[...], be1[...])
        hh = jnp.dot(h2, w2[...], preferred_element_type=jnp.float32)
        hs_o[...] = hh * dv[:, 0:1]

    return pl.pallas_call(
        body,
        out_shape=jax.ShapeDtypeStruct((NP, GOUT), jnp.float32),
    )(accg, bg, g1, beta1, W2, dinv8)


def _tc_d(S2, dinv8, b2, g2, beta2, x_p, W3, b3, W4, b4):
    def body(s2, dv, b2r, g2r, be2, xr, w3, b3r, w4, b4r, o):
        S = jnp.concatenate([s2[0], s2[1]], axis=1)
        h3 = S * dv[:, 0:1] + b2r[...]
        h3 = jnp.maximum(_ln(h3, g2r[...], be2[...]), 0.0)
        hc = jnp.concatenate([h3, xr[...]], axis=1)
        h4 = jnp.maximum(
            jnp.dot(hc, w3[...], preferred_element_type=jnp.float32) + b3r[...],
            0.0)
        o[...] = jnp.dot(h4, w4[...], preferred_element_type=jnp.float32) + b4r[...]

    return pl.pallas_call(
        body,
        out_shape=jax.ShapeDtypeStruct((NP, 2), jnp.float32),
    )(S2, dinv8, b2, g2, beta2, x_p, W3, b3, W4, b4)


def kernel(x, edge_index, W1, b1, Wg, att_src, att_dst, bg, g1, beta1,
           W2, b2, g2, beta2, W3, b3, W4, b4):
    loop = jnp.arange(N, dtype=jnp.int32)
    src = jnp.concatenate([edge_index[0], loop])
    dst = jnp.concatenate([edge_index[1], loop])
    pad = jnp.full((EP - ET,), NP - 1, jnp.int32)
    src_p = jnp.concatenate([src, pad])
    dst_p = jnp.concatenate([dst, pad])
    x_p = jnp.pad(x, ((0, NP - N), (0, 0)))

    val8 = jnp.concatenate(
        [jnp.ones((EB, 1), jnp.float32), jnp.zeros((EB, 7), jnp.float32)], axis=1)
    z8 = jnp.zeros((NP, 8), jnp.float32)
    z32 = jnp.zeros((NP, 32), jnp.float32)
    z64 = jnp.zeros((NP, 64), jnp.float32)
    z144 = jnp.zeros((NP, 144), jnp.float32)

    degp = _deg_call(dst_p, val8, z8)
    hs1, dinv8 = _tc_a(degp, x_p, W1)
    hs1f = jnp.concatenate([hs1[:, :32], hs1[:, 32:]], axis=0)
    S1 = _gcn_pass(hs1f, src_p, dst_p, z32, 32)

    head = jnp.arange(HEADS * HID, dtype=jnp.int32) // HID
    sel = (head[:, None] == jnp.arange(HEADS, dtype=jnp.int32)[None, :])
    As = jnp.where(sel, att_src.reshape(-1)[:, None], 0.0)
    Ad = jnp.where(sel, att_dst.reshape(-1)[:, None], 0.0)

    g, a_s, a_d, mp = _tc_b(S1, dinv8, b1, Wg, As, Ad)
    gf = jnp.concatenate([
        jnp.pad(jnp.concatenate([g[:, :128], a_s[:, 0:2]], axis=1),
                ((0, 0), (0, 14))),
        jnp.pad(jnp.concatenate([g[:, 128:], a_s[:, 2:4]], axis=1),
                ((0, 0), (0, 14))),
    ], axis=0)
    adst_t = jnp.concatenate([
        jnp.pad(jnp.concatenate([a_d[:, 0:2], mp[:, 0:2]], axis=1),
                ((0, 0), (0, 12))),
        jnp.pad(jnp.concatenate([a_d[:, 2:4], mp[:, 2:4]], axis=1),
                ((0, 0), (0, 12))),
    ], axis=0)
    accg = _gat_pass(gf, adst_t, src_p, dst_p, z144)

    hs2 = _tc_c(accg, bg, g1, beta1, W2, dinv8)
    hs2f = jnp.concatenate([hs2[:, :64], hs2[:, 64:]], axis=0)
    S2 = _gcn_pass(hs2f, src_p, dst_p, z64, 64)

    out = _tc_d(S2, dinv8, b2, g2, beta2, x_p, W3, b3, W4, b4)
    return out[:N]
